# Initial kernel scaffold; baseline (speedup 1.0000x reference)
import jax
import jax.numpy as jnp
from jax.experimental import pallas as pl


def kernel(input, table):
    raise NotImplementedError("write your pallas kernel here")


# speedup vs baseline: 1.0942x; 1.0942x over previous
import functools

import jax
import jax.numpy as jnp
from jax import lax
from jax.experimental import pallas as pl
from jax.experimental.pallas import tpu as pltpu
from jax.experimental.pallas import tpu_sc as plsc

B = 16384 * 50
D = 32
NC, NS = 2, 16
NW = NC * NS
BPW = B // NW
C = 1024
NCHUNK = BPW // C


def _gather_kernel(idx_hbm, table_hbm, out_hbm, idx_v, rows_v, sem):
    wid = lax.axis_index("s") * NC + lax.axis_index("c")
    base = wid * BPW

    def chunk(i, carry):
        off = base + i * C
        pltpu.sync_copy(idx_hbm.at[pl.ds(off, C)], idx_v)
        pltpu.async_copy(table_hbm.at[idx_v], rows_v, sem).wait()
        pltpu.sync_copy(rows_v, out_hbm.at[pl.ds(off, C)])
        return carry

    lax.fori_loop(0, NCHUNK, chunk, 0)


@jax.jit
def _gather(idx, table):
    mesh = plsc.VectorSubcoreMesh(core_axis_name="c", subcore_axis_name="s")
    f = pl.kernel(
        _gather_kernel,
        out_type=jax.ShapeDtypeStruct((B, D), jnp.float32),
        mesh=mesh,
        scratch_types=[
            pltpu.VMEM((C,), jnp.int32),
            pltpu.VMEM((C, D), jnp.float32),
            pltpu.SemaphoreType.DMA,
        ],
        compiler_params=pltpu.CompilerParams(use_tc_tiling_on_sc=False),
    )
    return f(idx, table)


def kernel(input, table):
    idx = input.reshape(-1).astype(jnp.int32)
    out = _gather(idx, table)
    return out.reshape(input.shape + (D,))


# speedup vs baseline: 1.1133x; 1.0175x over previous
import jax
import jax.numpy as jnp
from jax import lax
from jax.experimental import pallas as pl
from jax.experimental.pallas import tpu as pltpu
from jax.experimental.pallas import tpu_sc as plsc

B = 16384 * 50
D = 32
NC, NS = 2, 16
NW = NC * NS
BPW = B // NW
C = 640
NCHUNK = BPW // C
NBUF = 4
NGROUP = NCHUNK // NBUF
SHIFT = NBUF - 1


def _gather_kernel(idx_hbm, table_hbm, out_hbm, idx_v, rows_v, *sems):
    semg = sems[:NBUF]
    semw = sems[NBUF:]
    wid = lax.axis_index("s") * NC + lax.axis_index("c")
    base = wid * BPW

    pltpu.sync_copy(idx_hbm.at[pl.ds(base, BPW)], idx_v)

    def gather_desc(i, b):
        return pltpu.make_async_copy(
            table_hbm.at[idx_v.at[pl.ds(i * C, C)]], rows_v.at[b], semg[b])

    def wb_desc(i, b):
        return pltpu.make_async_copy(
            rows_v.at[b], out_hbm.at[pl.ds(base + i * C, C)], semw[b])

    for b in range(SHIFT):
        gather_desc(b, b).start()

    def group(g, carry):
        for b in range(NBUF):
            i1 = g * NBUF + b
            i2 = i1 + SHIFT
            b2 = (b + SHIFT) % NBUF

            @pl.when(i1 >= 1)
            def _():
                wb_desc(i1 - 1, b2).wait()

            @pl.when(i2 < NCHUNK)
            def _():
                gather_desc(i2, b2).start()

            gather_desc(i1, b).wait()
            wb_desc(i1, b).start()
        return carry

    lax.fori_loop(0, NGROUP, group, 0)
    wb_desc(NCHUNK - 1, (NCHUNK - 1) % NBUF).wait()


@jax.jit
def _gather(idx, table):
    mesh = plsc.VectorSubcoreMesh(core_axis_name="c", subcore_axis_name="s")
    f = pl.kernel(
        _gather_kernel,
        out_type=jax.ShapeDtypeStruct((B, D), jnp.float32),
        mesh=mesh,
        scratch_types=(
            [pltpu.VMEM((BPW,), jnp.int32),
             pltpu.VMEM((NBUF, C, D), jnp.float32)]
            + [pltpu.SemaphoreType.DMA] * (2 * NBUF)
        ),
        compiler_params=pltpu.CompilerParams(use_tc_tiling_on_sc=False),
    )
    return f(idx, table)


def kernel(input, table):
    idx = input.reshape(-1).astype(jnp.int32)
    out = _gather(idx, table)
    return out.reshape(input.shape + (D,))


# speedup vs baseline: 1.5967x; 1.4341x over previous
import jax
import jax.numpy as jnp
from jax import lax
from jax.experimental import pallas as pl
from jax.experimental.pallas import tpu as pltpu
from jax.experimental.pallas import tpu_sc as plsc

NJ = 50
NB = 16384
D = 32
NC, NS = 2, 16
NW = NC * NS
CB = NB // 128
NBLK = NJ * CB
BPW = NBLK // NW
NBUF = 4


def _gather_kernel(idx_hbm, table_hbm, out_hbm, idx_v, rows_v, t5_v, *sems):
    sem_i = sems[:NBUF]
    sem_g = sems[NBUF:2 * NBUF]
    sem_w = sems[2 * NBUF:]
    wid = lax.axis_index("s") * NC + lax.axis_index("c")
    t0 = wid * BPW

    iotas = [lax.iota(jnp.int32, 16) + (lg * 16) for lg in range(8)]

    def idx_desc(t, b):
        j = t // CB
        cb = t % CB
        return pltpu.make_async_copy(
            idx_hbm.at[j, pl.ds(cb * 128, 128)], idx_v.at[b], sem_i[b])

    def gather_desc(b):
        return pltpu.make_async_copy(
            table_hbm.at[idx_v.at[b]], rows_v.at[b], sem_g[b])

    def wb_descs(t, b):
        j = t // CB
        cb = t % CB
        return [pltpu.make_async_copy(
                    t5_v.at[b, pl.ds(r * 8, 8), :], out_hbm.at[j, r, cb],
                    sem_w[b])
                for r in range(4)]

    def transpose_block(b):
        for d in range(D):
            col = jnp.full((16,), d, jnp.int32)
            for lg in range(8):
                vec = plsc.load_gather(rows_v.at[b], [iotas[lg], col])
                t5_v[b, d, pl.ds(lg * 16, 16)] = vec

    for k in range(3):
        idx_desc(t0 + k, k).start()
    for k in range(2):
        idx_desc(t0 + k, k).wait()
        gather_desc(k).start()

    def group(g, carry):
        for b in range(NBUF):
            t = g * NBUF + b

            @pl.when(t + 3 < BPW)
            def _():
                idx_desc(t0 + t + 3, (b + 3) % NBUF).start()

            @pl.when(t + 2 < BPW)
            def _():
                idx_desc(t0 + t + 2, (b + 2) % NBUF).wait()
                gather_desc((b + 2) % NBUF).start()

            gather_desc(b).wait()

            @pl.when(t >= NBUF)
            def _():
                for d_ in wb_descs(t0 + t, b):
                    d_.wait()

            transpose_block(b)
            for d_ in wb_descs(t0 + t, b):
                d_.start()
        return carry

    lax.fori_loop(0, BPW // NBUF, group, 0)
    for b in range(NBUF):
        for d_ in wb_descs(t0 + BPW - NBUF + b, b):
            d_.wait()


@jax.jit
def _gather(idxT, table):
    mesh = plsc.VectorSubcoreMesh(core_axis_name="c", subcore_axis_name="s")
    f = pl.kernel(
        _gather_kernel,
        out_type=jax.ShapeDtypeStruct((NJ, 4, CB, 8, 128), jnp.float32),
        mesh=mesh,
        scratch_types=(
            [pltpu.VMEM((NBUF, 128), jnp.int32),
             pltpu.VMEM((NBUF, 128, D), jnp.float32),
             pltpu.VMEM((NBUF, D, 128), jnp.float32)]
            + [pltpu.SemaphoreType.DMA] * (3 * NBUF)
        ),
        compiler_params=pltpu.CompilerParams(use_tc_tiling_on_sc=False,
                                             needs_layout_passes=False),
    )
    return f(idxT, table)


def kernel(input, table):
    out5 = _gather(input.T, table)
    return out5.transpose(2, 4, 0, 1, 3).reshape(NB, NJ, D)


# speedup vs baseline: 1.9404x; 1.2153x over previous
import jax
import jax.numpy as jnp
from jax import lax
from jax.experimental import pallas as pl
from jax.experimental.pallas import tpu as pltpu
from jax.experimental.pallas import tpu_sc as plsc

NJ = 50
NB = 16384
B = NJ * NB
D = 32
NC, NS = 2, 16
NW = NC * NS
BPW = B // NW
C = 640
NCHUNK = BPW // C
NBUF = 4
NGROUP = NCHUNK // NBUF
SHIFT = NBUF - 1


def _gather_kernel(idx_hbm, table_hbm, out_hbm, idx_v, rows_v, *sems):
    semg = sems[:NBUF]
    semw = sems[NBUF:]
    wid = lax.axis_index("s") * NC + lax.axis_index("c")
    base = wid * BPW

    pltpu.sync_copy(idx_hbm.at[pl.ds(base, BPW)], idx_v)

    def gather_desc(i, b):
        return pltpu.make_async_copy(
            table_hbm.at[idx_v.at[pl.ds(i * C, C)]], rows_v.at[b], semg[b])

    def wb_desc(i, b):
        return pltpu.make_async_copy(
            rows_v.at[b], out_hbm.at[pl.ds(base + i * C, C)], semw[b])

    for b in range(SHIFT):
        gather_desc(b, b).start()

    def group(g, carry):
        for b in range(NBUF):
            i1 = g * NBUF + b
            i2 = i1 + SHIFT
            b2 = (b + SHIFT) % NBUF

            @pl.when(i1 >= 1)
            def _():
                wb_desc(i1 - 1, b2).wait()

            @pl.when(i2 < NCHUNK)
            def _():
                gather_desc(i2, b2).start()

            gather_desc(i1, b).wait()
            wb_desc(i1, b).start()
        return carry

    lax.fori_loop(0, NGROUP, group, 0)
    wb_desc(NCHUNK - 1, (NCHUNK - 1) % NBUF).wait()


@jax.jit
def _gather(idx, table):
    mesh = plsc.VectorSubcoreMesh(core_axis_name="c", subcore_axis_name="s")
    f = pl.kernel(
        _gather_kernel,
        out_type=jax.ShapeDtypeStruct((B, D), jnp.float32),
        mesh=mesh,
        scratch_types=(
            [pltpu.VMEM((BPW,), jnp.int32),
             pltpu.VMEM((NBUF, C, D), jnp.float32)]
            + [pltpu.SemaphoreType.DMA] * (2 * NBUF)
        ),
        compiler_params=pltpu.CompilerParams(use_tc_tiling_on_sc=False),
    )
    return f(idx, table)


def kernel(input, table):
    idx = input.T.reshape(-1)
    out = _gather(idx, table)
    return out.reshape(NJ, NB, D).transpose(1, 0, 2)
